# Initial kernel scaffold; baseline (speedup 1.0000x reference)
import jax
import jax.numpy as jnp
from jax.experimental import pallas as pl


def kernel(heat, K):
    raise NotImplementedError("write your pallas kernel here")


# speedup vs baseline: 3.0483x; 3.0483x over previous
import functools

import jax
import jax.numpy as jnp
from jax import lax
from jax.experimental import pallas as pl

_B, _C, _H, _W = 8, 17, 200, 200
_K = 40
_NEG = float("-inf")


def _nms_topk_body(x_ref, s_ref, i_ref, y_ref, xo_ref):
    v = x_ref[0, 0, :, :]
    negcol = jnp.full((_H, 1), _NEG, jnp.float32)
    left = jnp.concatenate([negcol, v[:, :-1]], axis=1)
    right = jnp.concatenate([v[:, 1:], negcol], axis=1)
    h = jnp.maximum(jnp.maximum(left, v), right)
    negrow = jnp.full((1, _W), _NEG, jnp.float32)
    up = jnp.concatenate([negrow, h[:-1, :]], axis=0)
    down = jnp.concatenate([h[1:, :], negrow], axis=0)
    hmax = jnp.maximum(jnp.maximum(up, h), down)
    w = jnp.where(v == hmax, v, 0.0)

    flat = (lax.broadcasted_iota(jnp.int32, (_H, _W), 0) * _W
            + lax.broadcasted_iota(jnp.int32, (_H, _W), 1))
    kiota = lax.broadcasted_iota(jnp.int32, (1, 128), 1)

    def body(k, carry):
        w, s_acc, i_acc = carry
        m = jnp.max(w)
        idx = jnp.min(jnp.where(w == m, flat, jnp.int32(1 << 30)))
        s_acc = jnp.where(kiota == k, m, s_acc)
        i_acc = jnp.where(kiota == k, idx, i_acc)
        w = jnp.where(flat == idx, _NEG, w)
        return (w, s_acc, i_acc)

    s0 = jnp.zeros((1, 128), jnp.float32)
    i0 = jnp.zeros((1, 128), jnp.int32)
    _, s_acc, i_acc = lax.fori_loop(0, _K, body, (w, s0, i0))

    s_ref[0, 0, 0, :] = s_acc[0, :_K]
    inds = i_acc[0, :_K]
    i_ref[0, 0, 0, :] = inds
    y_ref[0, 0, 0, :] = (inds // _W).astype(jnp.float32)
    xo_ref[0, 0, 0, :] = (inds % _W).astype(jnp.float32)


@jax.jit
def _nms_topk(heat):
    out = pl.pallas_call(
        _nms_topk_body,
        grid=(_B, _C),
        in_specs=[pl.BlockSpec((1, 1, _H, _W), lambda b, c: (b, c, 0, 0))],
        out_specs=[
            pl.BlockSpec((1, 1, 1, _K), lambda b, c: (b, c, 0, 0)),
            pl.BlockSpec((1, 1, 1, _K), lambda b, c: (b, c, 0, 0)),
            pl.BlockSpec((1, 1, 1, _K), lambda b, c: (b, c, 0, 0)),
            pl.BlockSpec((1, 1, 1, _K), lambda b, c: (b, c, 0, 0)),
        ],
        out_shape=[
            jax.ShapeDtypeStruct((_B, _C, 1, _K), jnp.float32),
            jax.ShapeDtypeStruct((_B, _C, 1, _K), jnp.int32),
            jax.ShapeDtypeStruct((_B, _C, 1, _K), jnp.float32),
            jax.ShapeDtypeStruct((_B, _C, 1, _K), jnp.float32),
        ],
    )(heat)
    return tuple(o.reshape(_B, _C, _K) for o in out)


def kernel(heat, K):
    del K
    return _nms_topk(heat)


# speedup vs baseline: 9.6390x; 3.1621x over previous
import functools

import jax
import jax.numpy as jnp
from jax import lax
from jax.experimental import pallas as pl
from jax.experimental.pallas import tpu as pltpu
from jax.experimental.pallas import tpu_sc as plsc

_B, _C, _H, _W = 8, 17, 200, 200
_K = 40
_R = _B * _C
_N = _H * _W
_CAP = 512
_NV = _CAP // 16
_NC, _NS = 2, 16
_NWORK = _NC * _NS
_ROWS_PER = -(-_R // _NWORK)
_PAD = 224
_NEG = float("-inf")
_BIG = 1 << 30


def _sc_body(heat, out_s, out_i, out_y, out_x,
             img, vrow, cand_v, cand_i, sel_v, sel_i, sel_y, sel_x):
    wid = lax.axis_index("s") * _NC + lax.axis_index("c")
    lane = lax.iota(jnp.int32, 16)
    neg16 = jnp.full((16,), _NEG, jnp.float32)

    def reduce_buffer():
        def mx(q, m):
            return jnp.maximum(m, cand_v[pl.ds(q * 16, 16)])
        maxv0 = lax.fori_loop(0, _NV, mx, neg16)

        def pick(k, carry):
            maxv, _ = carry
            m_val = jnp.max(maxv)

            def scanpos(q, best):
                vq = cand_v[pl.ds(q * 16, 16)]
                posq = jnp.min(jnp.where(vq == m_val, q * 16 + lane, jnp.int32(_BIG)))
                return jnp.minimum(best, posq)
            p = lax.fori_loop(0, _NV, scanpos, jnp.int32(_BIG))

            pv = jnp.full((16,), p, jnp.int32)
            lane0 = lane == 0
            iv = plsc.load_gather(cand_i, [pv])
            kv = jnp.full((16,), k, jnp.int32)
            plsc.store_scatter(sel_v, [kv], jnp.full((16,), m_val, jnp.float32),
                               mask=lane0)
            plsc.store_scatter(sel_i, [kv], iv, mask=lane0)
            plsc.store_scatter(cand_v, [pv], neg16, mask=lane0)
            lc = p % 16
            col = lane * 16 + lc
            g1 = plsc.load_gather(cand_v, [col])
            g2 = plsc.load_gather(cand_v, [col + 256])
            newm = jnp.maximum(jnp.max(g1), jnp.max(g2))
            maxv = jnp.where(lane == lc, newm, maxv)
            return maxv, m_val

        _, t_new = lax.fori_loop(0, _K, pick, (maxv0, jnp.float32(0.0)))

        def wipe(q, c):
            cand_v[pl.ds(q * 16, 16)] = neg16
            return c
        lax.fori_loop(0, _NV, wipe, 0)
        for m in range(3):
            cand_v[pl.ds(m * 16, 16)] = sel_v[pl.ds(m * 16, 16)]
            cand_i[pl.ds(m * 16, 16)] = sel_i[pl.ds(m * 16, 16)]
        return t_new

    def do_row(r):
        pltpu.sync_copy(heat.at[pl.ds(r * _N, _N)], img.at[pl.ds(_PAD, _N)])
        for m in range(_PAD // 16):
            img[pl.ds(m * 16, 16)] = neg16
            img[pl.ds(_PAD + _N + m * 16, 16)] = neg16
        for m in range(_NV):
            cand_v[pl.ds(m * 16, 16)] = neg16
        sel_v[pl.ds(32, 16)] = neg16

        def jbody(j, carry):
            t, cnt = carry
            t, cnt = lax.cond(
                cnt > _CAP - 224,
                lambda: (reduce_buffer(), jnp.int32(_K)),
                lambda: (t, cnt))

            vrow[pl.ds(0, 16)] = neg16

            def vert(ci, c):
                b = _PAD + j * 200 + ci * 16
                cm = jnp.maximum(
                    jnp.maximum(img[pl.ds(b - 200, 16)], img[pl.ds(b, 16)]),
                    img[pl.ds(b + 200, 16)])
                vrow[pl.ds(8 + ci * 16, 16)] = cm
                return c
            lax.fori_loop(0, 13, vert, 0)
            vrow[pl.ds(208, 16)] = neg16

            def hscan(ci, carry2):
                t2, cnt2 = carry2
                c0 = ci * 16
                ctr = vrow[pl.ds(8 + c0, 16)]
                lft = vrow[pl.ds(7 + c0, 16)]
                rgt = vrow[pl.ds(9 + c0, 16)]
                hm = jnp.maximum(jnp.maximum(lft, ctr), rgt)
                v = img[pl.ds(_PAD + j * 200 + c0, 16)]
                w = jnp.where(v == hm, v, jnp.float32(0.0))
                cols = c0 + lane
                mask = (w >= jnp.full((16,), t2, jnp.float32)) & (cols < 200)
                mi = mask.astype(jnp.int32)
                npass = jnp.sum(mi)

                def app(c):
                    pos = c - 1 + lax.cumsum(mi, axis=0)
                    plsc.store_scatter(cand_v, [pos], w, mask=mask)
                    plsc.store_scatter(cand_i, [pos], j * 200 + cols, mask=mask)
                    return c + npass
                cnt2 = lax.cond(npass > 0, app, lambda c: c, cnt2)
                return t2, cnt2

            return lax.fori_loop(0, 13, hscan, (t, cnt))

        lax.fori_loop(0, _H, jbody, (jnp.float32(0.0), jnp.int32(0)))

        reduce_buffer()
        for m in range(3):
            idx = sel_i[pl.ds(m * 16, 16)]
            y = idx // _W
            x = idx - y * _W
            sel_y[pl.ds(m * 16, 16)] = y.astype(jnp.float32)
            sel_x[pl.ds(m * 16, 16)] = x.astype(jnp.float32)
        pltpu.sync_copy(sel_v, out_s.at[pl.ds(r * 48, 48)])
        pltpu.sync_copy(sel_i, out_i.at[pl.ds(r * 48, 48)])
        pltpu.sync_copy(sel_y, out_y.at[pl.ds(r * 48, 48)])
        pltpu.sync_copy(sel_x, out_x.at[pl.ds(r * 48, 48)])

    def rloop(m, c):
        r = wid + m * _NWORK

        @pl.when(r < _R)
        def _():
            do_row(r)
        return c
    lax.fori_loop(0, _ROWS_PER, rloop, 0)


@jax.jit
def _sc_topk(heat2d):
    f32, i32 = jnp.float32, jnp.int32
    out = pl.kernel(
        _sc_body,
        out_type=[jax.ShapeDtypeStruct((_R * 48,), f32),
                  jax.ShapeDtypeStruct((_R * 48,), i32),
                  jax.ShapeDtypeStruct((_R * 48,), f32),
                  jax.ShapeDtypeStruct((_R * 48,), f32)],
        mesh=plsc.VectorSubcoreMesh(core_axis_name="c", subcore_axis_name="s"),
        compiler_params=pltpu.CompilerParams(needs_layout_passes=False),
        scratch_types=[pltpu.VMEM((_PAD + _N + _PAD,), f32),
                       pltpu.VMEM((224,), f32),
                       pltpu.VMEM((_CAP,), f32),
                       pltpu.VMEM((_CAP,), i32),
                       pltpu.VMEM((48,), f32),
                       pltpu.VMEM((48,), i32),
                       pltpu.VMEM((48,), f32),
                       pltpu.VMEM((48,), f32)],
    )(heat2d)
    return tuple(o.reshape(_R, 48)[:, :_K].reshape(_B, _C, _K) for o in out)


def kernel(heat, K):
    del K
    return _sc_topk(heat.reshape(_R * _N))


# speedup vs baseline: 14.3479x; 1.4885x over previous
import functools

import jax
import jax.numpy as jnp
from jax import lax
from jax.experimental import pallas as pl
from jax.experimental.pallas import tpu as pltpu
from jax.experimental.pallas import tpu_sc as plsc

_B, _C, _H, _W = 8, 17, 200, 200
_K = 40
_R = _B * _C
_N = _H * _W
_CAP = 512
_NV = _CAP // 16
_NC, _NS = 2, 16
_NWORK = _NC * _NS
_ROWS_PER = -(-_R // _NWORK)
_PAD = 224
_NEG = float("-inf")
_BIG = 1 << 30


def _sc_body(heat, out_s, out_i, out_y, out_x,
             img, vrow, rmax, cand_v, cand_i, sel_v, sel_i, sel_y, sel_x):
    wid = lax.axis_index("s") * _NC + lax.axis_index("c")
    lane = lax.iota(jnp.int32, 16)
    lane0 = lane == 0
    neg16 = jnp.full((16,), _NEG, jnp.float32)

    def reduce_buffer():
        def mx(q, carry):
            m, rowv = carry
            vq = cand_v[pl.ds(q * 16, 16)]
            upd = vq > m
            return jnp.maximum(m, vq), jnp.where(upd, q, rowv)
        maxv0, rowv0 = lax.fori_loop(0, _NV, mx,
                                     (neg16, jnp.zeros((16,), jnp.int32)))

        def pick(k, carry):
            maxv, rowv, _ = carry
            m_val = jnp.max(maxv)
            p = jnp.min(jnp.where(maxv == m_val, rowv * 16 + lane,
                                  jnp.int32(_BIG)))
            pv = jnp.full((16,), p, jnp.int32)
            iv = plsc.load_gather(cand_i, [pv])
            kv = jnp.full((16,), k, jnp.int32)
            plsc.store_scatter(sel_v, [kv],
                               jnp.full((16,), m_val, jnp.float32), mask=lane0)
            plsc.store_scatter(sel_i, [kv], iv, mask=lane0)
            plsc.store_scatter(cand_v, [pv], neg16, mask=lane0)
            lc = p % 16
            col = lane * 16 + lc
            g1 = plsc.load_gather(cand_v, [col])
            g2 = plsc.load_gather(cand_v, [col + 256])
            m1, m2 = jnp.max(g1), jnp.max(g2)
            newm = jnp.maximum(m1, m2)
            r1 = jnp.min(jnp.where(g1 == newm, lane, jnp.int32(_BIG)))
            r2 = jnp.min(jnp.where(g2 == newm, lane + 16, jnp.int32(_BIG)))
            newrow = jnp.where(m1 >= m2, r1, r2)
            sel = lane == lc
            return (jnp.where(sel, newm, maxv),
                    jnp.where(sel, newrow, rowv), m_val)

        _, _, t_new = lax.fori_loop(0, _K, pick,
                                    (maxv0, rowv0, jnp.float32(0.0)))

        def wipe(q, c):
            cand_v[pl.ds(q * 16, 16)] = neg16
            return c
        lax.fori_loop(0, _NV, wipe, 0)
        for m in range(3):
            cand_v[pl.ds(m * 16, 16)] = sel_v[pl.ds(m * 16, 16)]
            cand_i[pl.ds(m * 16, 16)] = sel_i[pl.ds(m * 16, 16)]
        return t_new

    def do_row(r):
        pltpu.sync_copy(heat.at[pl.ds(r * _N, _N)], img.at[pl.ds(_PAD, _N)])
        for m in range(_PAD // 16):
            img[pl.ds(m * 16, 16)] = neg16
            img[pl.ds(_PAD + _N + m * 16, 16)] = neg16
        for m in range(_NV):
            cand_v[pl.ds(m * 16, 16)] = neg16
        sel_v[pl.ds(32, 16)] = neg16

        def rmpass(j, c):
            b = _PAD + j * 200
            m = img[pl.ds(b, 16)]
            for ci in range(1, 12):
                m = jnp.maximum(m, img[pl.ds(b + ci * 16, 16)])
            tail = jnp.where(lane < 8, img[pl.ds(b + 192, 16)], neg16)
            m = jnp.maximum(m, tail)
            plsc.store_scatter(rmax, [jnp.full((16,), j, jnp.int32)],
                               jnp.full((16,), jnp.max(m), jnp.float32),
                               mask=lane0)
            return c
        lax.fori_loop(0, _H, rmpass, 0)

        def jbody(j, carry):
            t, cnt = carry
            t, cnt = lax.cond(
                cnt > _CAP - 224,
                lambda: (reduce_buffer(), jnp.int32(_K)),
                lambda: (t, cnt))

            rm = jnp.max(plsc.load_gather(rmax, [jnp.full((16,), j, jnp.int32)]))

            def process(cnt2):
                vrow[pl.ds(0, 16)] = neg16
                b = _PAD + j * 200
                for ci in range(13):
                    cm = jnp.maximum(
                        jnp.maximum(img[pl.ds(b - 200 + ci * 16, 16)],
                                    img[pl.ds(b + ci * 16, 16)]),
                        img[pl.ds(b + 200 + ci * 16, 16)])
                    vrow[pl.ds(8 + ci * 16, 16)] = cm
                vrow[pl.ds(208, 16)] = neg16

                for ci in range(13):
                    c0 = ci * 16
                    ctr = vrow[pl.ds(8 + c0, 16)]
                    lft = vrow[pl.ds(7 + c0, 16)]
                    rgt = vrow[pl.ds(9 + c0, 16)]
                    hm = jnp.maximum(jnp.maximum(lft, ctr), rgt)
                    v = img[pl.ds(b + c0, 16)]
                    w = jnp.where(v == hm, v, jnp.float32(0.0))
                    mask = w >= jnp.full((16,), t, jnp.float32)
                    if ci == 12:
                        mask = mask & (lane < 8)
                    mi = mask.astype(jnp.int32)
                    npass = jnp.sum(mi)

                    def app(c, mi=mi, mask=mask, w=w, c0=c0, npass=npass):
                        pos = c - 1 + lax.cumsum(mi, axis=0)
                        plsc.store_scatter(cand_v, [pos], w, mask=mask)
                        plsc.store_scatter(cand_i, [pos],
                                           j * 200 + c0 + lane, mask=mask)
                        return c + npass
                    cnt2 = lax.cond(npass > 0, app, lambda c: c, cnt2)
                return cnt2

            cnt = lax.cond(rm >= t, process, lambda c: c, cnt)
            return t, cnt

        lax.fori_loop(0, _H, jbody, (jnp.float32(0.0), jnp.int32(0)))

        reduce_buffer()
        for m in range(3):
            idx = sel_i[pl.ds(m * 16, 16)]
            y = idx // _W
            x = idx - y * _W
            sel_y[pl.ds(m * 16, 16)] = y.astype(jnp.float32)
            sel_x[pl.ds(m * 16, 16)] = x.astype(jnp.float32)
        pltpu.sync_copy(sel_v, out_s.at[pl.ds(r * 48, 48)])
        pltpu.sync_copy(sel_i, out_i.at[pl.ds(r * 48, 48)])
        pltpu.sync_copy(sel_y, out_y.at[pl.ds(r * 48, 48)])
        pltpu.sync_copy(sel_x, out_x.at[pl.ds(r * 48, 48)])

    def rloop(m, c):
        r = wid + m * _NWORK

        @pl.when(r < _R)
        def _():
            do_row(r)
        return c
    lax.fori_loop(0, _ROWS_PER, rloop, 0)


@jax.jit
def _sc_topk(heat1d):
    f32, i32 = jnp.float32, jnp.int32
    out = pl.kernel(
        _sc_body,
        out_type=[jax.ShapeDtypeStruct((_R * 48,), f32),
                  jax.ShapeDtypeStruct((_R * 48,), i32),
                  jax.ShapeDtypeStruct((_R * 48,), f32),
                  jax.ShapeDtypeStruct((_R * 48,), f32)],
        mesh=plsc.VectorSubcoreMesh(core_axis_name="c", subcore_axis_name="s"),
        compiler_params=pltpu.CompilerParams(needs_layout_passes=False),
        scratch_types=[pltpu.VMEM((_PAD + _N + _PAD,), f32),
                       pltpu.VMEM((224,), f32),
                       pltpu.VMEM((208,), f32),
                       pltpu.VMEM((_CAP,), f32),
                       pltpu.VMEM((_CAP,), i32),
                       pltpu.VMEM((48,), f32),
                       pltpu.VMEM((48,), i32),
                       pltpu.VMEM((48,), f32),
                       pltpu.VMEM((48,), f32)],
    )(heat1d)
    return tuple(o.reshape(_R, 48)[:, :_K].reshape(_B, _C, _K) for o in out)


def kernel(heat, K):
    del K
    return _sc_topk(heat.reshape(_R * _N))


# speedup vs baseline: 17.7598x; 1.2378x over previous
import functools

import jax
import jax.numpy as jnp
from jax import lax
from jax.experimental import pallas as pl
from jax.experimental.pallas import tpu as pltpu
from jax.experimental.pallas import tpu_sc as plsc

_B, _C, _H, _W = 8, 17, 200, 200
_K = 40
_R = _B * _C
_N = _H * _W
_CAP = 512
_NV = _CAP // 16
_NC, _NS = 2, 16
_NWORK = _NC * _NS
_ROWS_PER = -(-_R // _NWORK)
_PAD = 224
_NEG = float("-inf")
_BIG = 1 << 30


def _sc_body(heat, out_s, out_i, out_y, out_x,
             img, vrow, rmax, cand_v, cand_i, sel_v, sel_i, sel_y, sel_x):
    wid = lax.axis_index("s") * _NC + lax.axis_index("c")
    lane = lax.iota(jnp.int32, 16)
    lane0 = lane == 0
    neg16 = jnp.full((16,), _NEG, jnp.float32)

    def reduce_buffer():
        def mx(q, carry):
            m, rowv = carry
            vq = cand_v[pl.ds(q * 16, 16)]
            upd = vq > m
            return jnp.maximum(m, vq), jnp.where(upd, q, rowv)
        maxv0, rowv0 = lax.fori_loop(0, _NV, mx,
                                     (neg16, jnp.zeros((16,), jnp.int32)))

        def pick(k, carry):
            maxv, rowv, _ = carry
            m_val = jnp.max(maxv)
            p = jnp.min(jnp.where(maxv == m_val, rowv * 16 + lane,
                                  jnp.int32(_BIG)))
            pv = jnp.full((16,), p, jnp.int32)
            iv = plsc.load_gather(cand_i, [pv])
            kv = jnp.full((16,), k, jnp.int32)
            plsc.store_scatter(sel_v, [kv],
                               jnp.full((16,), m_val, jnp.float32), mask=lane0)
            plsc.store_scatter(sel_i, [kv], iv, mask=lane0)
            plsc.store_scatter(cand_v, [pv], neg16, mask=lane0)
            lc = p % 16
            col = lane * 16 + lc
            g1 = plsc.load_gather(cand_v, [col])
            g2 = plsc.load_gather(cand_v, [col + 256])
            m1, m2 = jnp.max(g1), jnp.max(g2)
            newm = jnp.maximum(m1, m2)
            r1 = jnp.min(jnp.where(g1 == newm, lane, jnp.int32(_BIG)))
            r2 = jnp.min(jnp.where(g2 == newm, lane + 16, jnp.int32(_BIG)))
            newrow = jnp.where(m1 >= m2, r1, r2)
            sel = lane == lc
            return (jnp.where(sel, newm, maxv),
                    jnp.where(sel, newrow, rowv), m_val)

        _, _, t_new = lax.fori_loop(0, _K, pick,
                                    (maxv0, rowv0, jnp.float32(0.0)))

        def wipe(q, c):
            cand_v[pl.ds(q * 16, 16)] = neg16
            return c
        lax.fori_loop(0, _NV, wipe, 0)
        for m in range(3):
            cand_v[pl.ds(m * 16, 16)] = sel_v[pl.ds(m * 16, 16)]
            cand_i[pl.ds(m * 16, 16)] = sel_i[pl.ds(m * 16, 16)]
        return t_new

    def do_row(r):
        pltpu.sync_copy(heat.at[pl.ds(r * _N, _N)], img.at[pl.ds(_PAD, _N)])
        for m in range(_PAD // 16):
            img[pl.ds(m * 16, 16)] = neg16
            img[pl.ds(_PAD + _N + m * 16, 16)] = neg16
        for m in range(_NV):
            cand_v[pl.ds(m * 16, 16)] = neg16
        sel_v[pl.ds(32, 16)] = neg16

        def rmpass(j, c):
            b = _PAD + j * 200
            m = img[pl.ds(b, 16)]
            for ci in range(1, 12):
                m = jnp.maximum(m, img[pl.ds(b + ci * 16, 16)])
            tail = jnp.where(lane < 8, img[pl.ds(b + 192, 16)], neg16)
            m = jnp.maximum(m, tail)
            plsc.store_scatter(rmax, [jnp.full((16,), j, jnp.int32)],
                               jnp.full((16,), jnp.max(m), jnp.float32),
                               mask=lane0)
            return c
        lax.fori_loop(0, _H, rmpass, 0)

        def jbody(j, carry):
            t, cnt_v = carry
            cnt_s = jnp.max(cnt_v)
            t, cnt_v = lax.cond(
                cnt_s > _CAP - 224,
                lambda: (reduce_buffer(), jnp.full((16,), _K, jnp.int32)),
                lambda: (t, cnt_v))

            rm = jnp.max(plsc.load_gather(rmax, [jnp.full((16,), j, jnp.int32)]))

            def process(cnt2):
                vrow[pl.ds(0, 16)] = neg16
                b = _PAD + j * 200
                ctrs = []
                for ci in range(13):
                    mid = img[pl.ds(b + ci * 16, 16)]
                    cm = jnp.maximum(
                        jnp.maximum(img[pl.ds(b - 200 + ci * 16, 16)], mid),
                        img[pl.ds(b + 200 + ci * 16, 16)])
                    vrow[pl.ds(8 + ci * 16, 16)] = cm
                    ctrs.append((mid, cm))
                vrow[pl.ds(208, 16)] = neg16

                t_v = jnp.full((16,), t, jnp.float32)
                for ci in range(13):
                    c0 = ci * 16
                    v, ctr = ctrs[ci]
                    lft = vrow[pl.ds(7 + c0, 16)]
                    rgt = vrow[pl.ds(9 + c0, 16)]
                    hm = jnp.maximum(jnp.maximum(lft, ctr), rgt)
                    w = jnp.where(v == hm, v, jnp.float32(0.0))
                    mask = w >= t_v
                    if ci == 12:
                        mask = mask & (lane < 8)
                    npass_v = plsc.all_reduce_population_count(mask)
                    pos = cnt2 - 1 + lax.cumsum(mask.astype(jnp.int32), axis=0)
                    plsc.store_scatter(cand_v, [pos], w, mask=mask)
                    plsc.store_scatter(cand_i, [pos],
                                       j * 200 + c0 + lane, mask=mask)
                    cnt2 = cnt2 + npass_v
                return cnt2

            cnt_v = lax.cond(rm >= t, process, lambda c: c, cnt_v)
            return t, cnt_v

        lax.fori_loop(0, _H, jbody,
                      (jnp.float32(0.0), jnp.zeros((16,), jnp.int32)))

        reduce_buffer()
        for m in range(3):
            idx = sel_i[pl.ds(m * 16, 16)]
            y = idx // _W
            x = idx - y * _W
            sel_y[pl.ds(m * 16, 16)] = y.astype(jnp.float32)
            sel_x[pl.ds(m * 16, 16)] = x.astype(jnp.float32)
        pltpu.sync_copy(sel_v, out_s.at[pl.ds(r * 48, 48)])
        pltpu.sync_copy(sel_i, out_i.at[pl.ds(r * 48, 48)])
        pltpu.sync_copy(sel_y, out_y.at[pl.ds(r * 48, 48)])
        pltpu.sync_copy(sel_x, out_x.at[pl.ds(r * 48, 48)])

    def rloop(m, c):
        r = wid + m * _NWORK

        @pl.when(r < _R)
        def _():
            do_row(r)
        return c
    lax.fori_loop(0, _ROWS_PER, rloop, 0)


@jax.jit
def _sc_topk(heat1d):
    f32, i32 = jnp.float32, jnp.int32
    out = pl.kernel(
        _sc_body,
        out_type=[jax.ShapeDtypeStruct((_R * 48,), f32),
                  jax.ShapeDtypeStruct((_R * 48,), i32),
                  jax.ShapeDtypeStruct((_R * 48,), f32),
                  jax.ShapeDtypeStruct((_R * 48,), f32)],
        mesh=plsc.VectorSubcoreMesh(core_axis_name="c", subcore_axis_name="s"),
        compiler_params=pltpu.CompilerParams(needs_layout_passes=False),
        scratch_types=[pltpu.VMEM((_PAD + _N + _PAD,), f32),
                       pltpu.VMEM((224,), f32),
                       pltpu.VMEM((208,), f32),
                       pltpu.VMEM((_CAP,), f32),
                       pltpu.VMEM((_CAP,), i32),
                       pltpu.VMEM((48,), f32),
                       pltpu.VMEM((48,), i32),
                       pltpu.VMEM((48,), f32),
                       pltpu.VMEM((48,), f32)],
    )(heat1d)
    return tuple(o.reshape(_R, 48)[:, :_K].reshape(_B, _C, _K) for o in out)


def kernel(heat, K):
    del K
    return _sc_topk(heat.reshape(_R * _N))


# speedup vs baseline: 19.4065x; 1.0927x over previous
import functools

import jax
import jax.numpy as jnp
from jax import lax
from jax.experimental import pallas as pl
from jax.experimental.pallas import tpu as pltpu
from jax.experimental.pallas import tpu_sc as plsc

_B, _C, _H, _W = 8, 17, 200, 200
_K = 40
_R = _B * _C
_N = _H * _W
_CAP = 512
_NV = _CAP // 16
_NC, _NS = 2, 16
_NWORK = _NC * _NS
_ROWS_PER = -(-_R // _NWORK)
_PAD = 224
_NEG = float("-inf")
_BIG = 1 << 30


def _sc_body(heat, out_s, out_i, out_y, out_x,
             img, vrow, rmax, cand_v, cand_i, sel_v, sel_i, sel_y, sel_x):
    wid = lax.axis_index("s") * _NC + lax.axis_index("c")
    lane = lax.iota(jnp.int32, 16)
    lane0 = lane == 0
    neg16 = jnp.full((16,), _NEG, jnp.float32)

    def reduce_buffer():
        def mx(q, carry):
            m, rowv = carry
            vq = cand_v[pl.ds(q * 16, 16)]
            upd = vq > m
            return jnp.maximum(m, vq), jnp.where(upd, q, rowv)
        maxv0, rowv0 = lax.fori_loop(0, _NV, mx,
                                     (neg16, jnp.zeros((16,), jnp.int32)))

        def pick(k, carry):
            maxv, rowv, _ = carry
            m_val = jnp.max(maxv)
            p = jnp.min(jnp.where(maxv == m_val, rowv * 16 + lane,
                                  jnp.int32(_BIG)))
            pv = jnp.full((16,), p, jnp.int32)
            iv = plsc.load_gather(cand_i, [pv])
            kv = jnp.full((16,), k, jnp.int32)
            plsc.store_scatter(sel_v, [kv],
                               jnp.full((16,), m_val, jnp.float32), mask=lane0)
            plsc.store_scatter(sel_i, [kv], iv, mask=lane0)
            plsc.store_scatter(cand_v, [pv], neg16, mask=lane0)
            lc = p % 16
            col = lane * 16 + lc
            g1 = plsc.load_gather(cand_v, [col])
            g2 = plsc.load_gather(cand_v, [col + 256])
            m1, m2 = jnp.max(g1), jnp.max(g2)
            newm = jnp.maximum(m1, m2)
            r1 = jnp.min(jnp.where(g1 == newm, lane, jnp.int32(_BIG)))
            r2 = jnp.min(jnp.where(g2 == newm, lane + 16, jnp.int32(_BIG)))
            newrow = jnp.where(m1 >= m2, r1, r2)
            sel = lane == lc
            return (jnp.where(sel, newm, maxv),
                    jnp.where(sel, newrow, rowv), m_val)

        _, _, t_new = lax.fori_loop(0, _K, pick,
                                    (maxv0, rowv0, jnp.float32(0.0)))

        def wipe(q, c):
            cand_v[pl.ds(q * 16, 16)] = neg16
            return c
        lax.fori_loop(0, _NV, wipe, 0)
        for m in range(3):
            cand_v[pl.ds(m * 16, 16)] = sel_v[pl.ds(m * 16, 16)]
            cand_i[pl.ds(m * 16, 16)] = sel_i[pl.ds(m * 16, 16)]
        return t_new

    def do_row(r):
        pltpu.sync_copy(heat.at[pl.ds(r * _N, _N)], img.at[pl.ds(_PAD, _N)])
        for m in range(_PAD // 16):
            img[pl.ds(m * 16, 16)] = neg16
            img[pl.ds(_PAD + _N + m * 16, 16)] = neg16
        for m in range(_NV):
            cand_v[pl.ds(m * 16, 16)] = neg16
        sel_v[pl.ds(32, 16)] = neg16

        def rmpass(j, c):
            b = _PAD + j * 200
            m = img[pl.ds(b, 16)]
            for ci in range(1, 12):
                m = jnp.maximum(m, img[pl.ds(b + ci * 16, 16)])
            tail = jnp.where(lane < 8, img[pl.ds(b + 192, 16)], neg16)
            m = jnp.maximum(m, tail)
            rmax[j] = jnp.max(m)
            return c
        lax.fori_loop(0, _H, rmpass, 0)

        def jbody(j, carry):
            t, cnt_v = carry

            def process():
                cnt_s = jnp.max(cnt_v)
                t2, cnt2 = lax.cond(
                    cnt_s > _CAP - 224,
                    lambda: (reduce_buffer(), jnp.full((16,), _K, jnp.int32)),
                    lambda: (t, cnt_v))
                vrow[pl.ds(0, 16)] = neg16
                b = _PAD + j * 200
                ctrs = []
                for ci in range(13):
                    mid = img[pl.ds(b + ci * 16, 16)]
                    cm = jnp.maximum(
                        jnp.maximum(img[pl.ds(b - 200 + ci * 16, 16)], mid),
                        img[pl.ds(b + 200 + ci * 16, 16)])
                    vrow[pl.ds(8 + ci * 16, 16)] = cm
                    ctrs.append((mid, cm))
                vrow[pl.ds(208, 16)] = neg16

                t_v = jnp.full((16,), t2, jnp.float32)
                for ci in range(13):
                    c0 = ci * 16
                    v, ctr = ctrs[ci]
                    lft = vrow[pl.ds(7 + c0, 16)]
                    rgt = vrow[pl.ds(9 + c0, 16)]
                    hm = jnp.maximum(jnp.maximum(lft, ctr), rgt)
                    w = jnp.where(v == hm, v, jnp.float32(0.0))
                    mask = w >= t_v
                    if ci == 12:
                        mask = mask & (lane < 8)
                    npass_v = plsc.all_reduce_population_count(mask)
                    pos = cnt2 - 1 + lax.cumsum(mask.astype(jnp.int32), axis=0)
                    plsc.store_scatter(cand_v, [pos], w, mask=mask)
                    plsc.store_scatter(cand_i, [pos],
                                       j * 200 + c0 + lane, mask=mask)
                    cnt2 = cnt2 + npass_v
                return t2, cnt2

            return lax.cond(rmax[j] >= t, process, lambda: (t, cnt_v))

        lax.fori_loop(0, _H, jbody,
                      (jnp.float32(0.0), jnp.zeros((16,), jnp.int32)))

        reduce_buffer()
        for m in range(3):
            idx = sel_i[pl.ds(m * 16, 16)]
            y = idx // _W
            x = idx - y * _W
            sel_y[pl.ds(m * 16, 16)] = y.astype(jnp.float32)
            sel_x[pl.ds(m * 16, 16)] = x.astype(jnp.float32)
        pltpu.sync_copy(sel_v.at[pl.ds(0, _K)], out_s.at[pl.ds(r * _K, _K)])
        pltpu.sync_copy(sel_i.at[pl.ds(0, _K)], out_i.at[pl.ds(r * _K, _K)])
        pltpu.sync_copy(sel_y.at[pl.ds(0, _K)], out_y.at[pl.ds(r * _K, _K)])
        pltpu.sync_copy(sel_x.at[pl.ds(0, _K)], out_x.at[pl.ds(r * _K, _K)])

    def rloop(m, c):
        r = wid + m * _NWORK

        @pl.when(r < _R)
        def _():
            do_row(r)
        return c
    lax.fori_loop(0, _ROWS_PER, rloop, 0)


@jax.jit
def _sc_topk(heat1d):
    f32, i32 = jnp.float32, jnp.int32
    out = pl.kernel(
        _sc_body,
        out_type=[jax.ShapeDtypeStruct((_R * _K,), f32),
                  jax.ShapeDtypeStruct((_R * _K,), i32),
                  jax.ShapeDtypeStruct((_R * _K,), f32),
                  jax.ShapeDtypeStruct((_R * _K,), f32)],
        mesh=plsc.VectorSubcoreMesh(core_axis_name="c", subcore_axis_name="s"),
        compiler_params=pltpu.CompilerParams(needs_layout_passes=False),
        scratch_types=[pltpu.VMEM((_PAD + _N + _PAD,), f32),
                       pltpu.VMEM((224,), f32),
                       pltpu.SMEM((208,), f32),
                       pltpu.VMEM((_CAP,), f32),
                       pltpu.VMEM((_CAP,), i32),
                       pltpu.VMEM((48,), f32),
                       pltpu.VMEM((48,), i32),
                       pltpu.VMEM((48,), f32),
                       pltpu.VMEM((48,), f32)],
    )(heat1d)
    return tuple(o.reshape(_B, _C, _K) for o in out)


def kernel(heat, K):
    del K
    return _sc_topk(heat.reshape(_R * _N))


# speedup vs baseline: 19.7290x; 1.0166x over previous
import functools

import jax
import jax.numpy as jnp
from jax import lax
from jax.experimental import pallas as pl
from jax.experimental.pallas import tpu as pltpu
from jax.experimental.pallas import tpu_sc as plsc

_B, _C, _H, _W = 8, 17, 200, 200
_K = 40
_R = _B * _C
_N = _H * _W
_CAP = 512
_NV = _CAP // 16
_NC, _NS = 2, 16
_NWORK = _NC * _NS
_ROWS_PER = -(-_R // _NWORK)
_PAD = 224
_NEG = float("-inf")
_BIG = 1 << 30


def _sc_body(heat, out_s, out_i, out_y, out_x,
             img, vrow, rmax, cand_v, cand_i, sel_v, sel_i, sel_y, sel_x, sem):
    wid = lax.axis_index("s") * _NC + lax.axis_index("c")
    lane = lax.iota(jnp.int32, 16)
    lane0 = lane == 0
    neg16 = jnp.full((16,), _NEG, jnp.float32)

    def reduce_buffer():
        def mx(q, carry):
            m, rowv = carry
            vq = cand_v[pl.ds(q * 16, 16)]
            upd = vq > m
            return jnp.maximum(m, vq), jnp.where(upd, q, rowv)
        maxv0, rowv0 = lax.fori_loop(0, _NV, mx,
                                     (neg16, jnp.zeros((16,), jnp.int32)))

        def pick(k, carry):
            maxv, rowv, _ = carry
            m_val = jnp.max(maxv)
            p = jnp.min(jnp.where(maxv == m_val, rowv * 16 + lane,
                                  jnp.int32(_BIG)))
            pv = jnp.full((16,), p, jnp.int32)
            iv = plsc.load_gather(cand_i, [pv])
            kv = jnp.full((16,), k, jnp.int32)
            plsc.store_scatter(sel_v, [kv],
                               jnp.full((16,), m_val, jnp.float32), mask=lane0)
            plsc.store_scatter(sel_i, [kv], iv, mask=lane0)
            plsc.store_scatter(cand_v, [pv], neg16, mask=lane0)
            lc = p % 16
            col = lane * 16 + lc
            g1 = plsc.load_gather(cand_v, [col])
            g2 = plsc.load_gather(cand_v, [col + 256])
            m1, m2 = jnp.max(g1), jnp.max(g2)
            newm = jnp.maximum(m1, m2)
            r1 = jnp.min(jnp.where(g1 == newm, lane, jnp.int32(_BIG)))
            r2 = jnp.min(jnp.where(g2 == newm, lane + 16, jnp.int32(_BIG)))
            newrow = jnp.where(m1 >= m2, r1, r2)
            sel = lane == lc
            return (jnp.where(sel, newm, maxv),
                    jnp.where(sel, newrow, rowv), m_val)

        _, _, t_new = lax.fori_loop(0, _K, pick,
                                    (maxv0, rowv0, jnp.float32(0.0)))

        def wipe(q, c):
            cand_v[pl.ds(q * 16, 16)] = neg16
            return c
        lax.fori_loop(0, _NV, wipe, 0)
        for m in range(3):
            cand_v[pl.ds(m * 16, 16)] = sel_v[pl.ds(m * 16, 16)]
            cand_i[pl.ds(m * 16, 16)] = sel_i[pl.ds(m * 16, 16)]
        return t_new

    _IMGW = _PAD + _N + _PAD

    def do_row(r, base):
        for m in range(_NV):
            cand_v[pl.ds(m * 16, 16)] = neg16
        sel_v[pl.ds(32, 16)] = neg16

        def rmpass(j, c):
            b = base + _PAD + j * 200
            m = img[pl.ds(b, 16)]
            for ci in range(1, 12):
                m = jnp.maximum(m, img[pl.ds(b + ci * 16, 16)])
            tail = jnp.where(lane < 8, img[pl.ds(b + 192, 16)], neg16)
            m = jnp.maximum(m, tail)
            rmax[j] = jnp.max(m)
            return c
        lax.fori_loop(0, _H, rmpass, 0)

        def jbody(j, carry):
            t, cnt_v = carry

            def process():
                cnt_s = jnp.max(cnt_v)
                t2, cnt2 = lax.cond(
                    cnt_s > _CAP - 224,
                    lambda: (reduce_buffer(), jnp.full((16,), _K, jnp.int32)),
                    lambda: (t, cnt_v))
                vrow[pl.ds(0, 16)] = neg16
                b = base + _PAD + j * 200
                ctrs = []
                for ci in range(13):
                    mid = img[pl.ds(b + ci * 16, 16)]
                    cm = jnp.maximum(
                        jnp.maximum(img[pl.ds(b - 200 + ci * 16, 16)], mid),
                        img[pl.ds(b + 200 + ci * 16, 16)])
                    vrow[pl.ds(8 + ci * 16, 16)] = cm
                    ctrs.append((mid, cm))
                vrow[pl.ds(208, 16)] = neg16

                t_v = jnp.full((16,), t2, jnp.float32)

                def hs(zero_phase, cnt3):
                    for ci in range(13):
                        c0 = ci * 16
                        v, ctr = ctrs[ci]
                        lft = vrow[pl.ds(7 + c0, 16)]
                        rgt = vrow[pl.ds(9 + c0, 16)]
                        hm = jnp.maximum(jnp.maximum(lft, ctr), rgt)
                        if zero_phase:
                            val = jnp.where(v == hm, v, jnp.float32(0.0))
                            mask = val >= t_v
                        else:
                            val = v
                            mask = v >= jnp.maximum(hm, t_v)
                        if ci == 12:
                            mask = mask & (lane < 8)
                        npass_v = plsc.all_reduce_population_count(mask)
                        pos = cnt3 - 1 + lax.cumsum(mask.astype(jnp.int32),
                                                    axis=0)
                        plsc.store_scatter(cand_v, [pos], val, mask=mask)
                        plsc.store_scatter(cand_i, [pos],
                                           j * 200 + c0 + lane, mask=mask)
                        cnt3 = cnt3 + npass_v
                    return cnt3

                cnt2 = lax.cond(t2 > 0,
                                lambda c: hs(False, c),
                                lambda c: hs(True, c), cnt2)
                return t2, cnt2

            return lax.cond(rmax[j] >= t, process, lambda: (t, cnt_v))

        lax.fori_loop(0, _H, jbody,
                      (jnp.float32(0.0), jnp.zeros((16,), jnp.int32)))

        reduce_buffer()
        for m in range(3):
            idx = sel_i[pl.ds(m * 16, 16)]
            y = idx // _W
            x = idx - y * _W
            sel_y[pl.ds(m * 16, 16)] = y.astype(jnp.float32)
            sel_x[pl.ds(m * 16, 16)] = x.astype(jnp.float32)
        pltpu.sync_copy(sel_v.at[pl.ds(0, _K)], out_s.at[pl.ds(r * _K, _K)])
        pltpu.sync_copy(sel_i.at[pl.ds(0, _K)], out_i.at[pl.ds(r * _K, _K)])
        pltpu.sync_copy(sel_y.at[pl.ds(0, _K)], out_y.at[pl.ds(r * _K, _K)])
        pltpu.sync_copy(sel_x.at[pl.ds(0, _K)], out_x.at[pl.ds(r * _K, _K)])

    for m in range(_PAD // 16):
        for base in (0, _IMGW):
            img[pl.ds(base + m * 16, 16)] = neg16
            img[pl.ds(base + _PAD + _N + m * 16, 16)] = neg16
    pltpu.async_copy(heat.at[pl.ds(wid * _N, _N)],
                     img.at[pl.ds(_PAD, _N)], sem.at[0])

    def rloop(m, c):
        r = wid + m * _NWORK

        @pl.when(r < _R)
        def _():
            cur = m % 2
            base = cur * _IMGW
            pltpu.make_async_copy(heat.at[pl.ds(r * _N, _N)],
                                  img.at[pl.ds(base + _PAD, _N)],
                                  sem.at[cur]).wait()
            rn = r + _NWORK

            @pl.when(rn < _R)
            def _():
                pltpu.async_copy(heat.at[pl.ds(rn * _N, _N)],
                                 img.at[pl.ds((_IMGW - base) + _PAD, _N)],
                                 sem.at[1 - cur])
            do_row(r, base)
        return c
    lax.fori_loop(0, _ROWS_PER, rloop, 0)


@jax.jit
def _sc_topk(heat1d):
    f32, i32 = jnp.float32, jnp.int32
    out = pl.kernel(
        _sc_body,
        out_type=[jax.ShapeDtypeStruct((_R * _K,), f32),
                  jax.ShapeDtypeStruct((_R * _K,), i32),
                  jax.ShapeDtypeStruct((_R * _K,), f32),
                  jax.ShapeDtypeStruct((_R * _K,), f32)],
        mesh=plsc.VectorSubcoreMesh(core_axis_name="c", subcore_axis_name="s"),
        compiler_params=pltpu.CompilerParams(needs_layout_passes=False),
        scratch_types=[pltpu.VMEM((2 * (_PAD + _N + _PAD),), f32),
                       pltpu.VMEM((224,), f32),
                       pltpu.SMEM((208,), f32),
                       pltpu.VMEM((_CAP,), f32),
                       pltpu.VMEM((_CAP,), i32),
                       pltpu.VMEM((48,), f32),
                       pltpu.VMEM((48,), i32),
                       pltpu.VMEM((48,), f32),
                       pltpu.VMEM((48,), f32),
                       pltpu.SemaphoreType.DMA((2,))],
    )(heat1d)
    return tuple(o.reshape(_B, _C, _K) for o in out)


def kernel(heat, K):
    del K
    return _sc_topk(heat.reshape(_R * _N))


# speedup vs baseline: 20.5735x; 1.0428x over previous
import functools

import jax
import jax.numpy as jnp
from jax import lax
from jax.experimental import pallas as pl
from jax.experimental.pallas import tpu as pltpu
from jax.experimental.pallas import tpu_sc as plsc

_B, _C, _H, _W = 8, 17, 200, 200
_K = 40
_R = _B * _C
_N = _H * _W
_CAP = 512
_NV = _CAP // 16
_NC, _NS = 2, 16
_NWORK = _NC * _NS
_FULL = 4
_PAD = 224
_IMGW = _PAD + _N + _PAD
_NEG = float("-inf")
_BIG = 1 << 30


def _sc_body(heat, out_s, out_i, out_y, out_x,
             img, vrow, rmax, cand_v, cand_i, sel_v, sel_i, sel_y, sel_x,
             spart_v, spart_i, sem):
    sidx = lax.axis_index("s")
    cidx = lax.axis_index("c")
    wid = sidx * _NC + cidx
    lane = lax.iota(jnp.int32, 16)
    lane0 = lane == 0
    neg16 = jnp.full((16,), _NEG, jnp.float32)

    def reduce_buffer(nv):
        def mx(q, carry):
            m, rowv = carry
            vq = cand_v[pl.ds(q * 16, 16)]
            upd = vq > m
            return jnp.maximum(m, vq), jnp.where(upd, q, rowv)
        maxv0, rowv0 = lax.fori_loop(0, nv, mx,
                                     (neg16, jnp.zeros((16,), jnp.int32)))

        def pick(k, carry):
            maxv, rowv, _ = carry
            m_val = jnp.max(maxv)
            p = jnp.min(jnp.where(maxv == m_val, rowv * 16 + lane,
                                  jnp.int32(_BIG)))
            pv = jnp.full((16,), p, jnp.int32)
            iv = plsc.load_gather(cand_i, [pv])
            kv = jnp.full((16,), k, jnp.int32)
            plsc.store_scatter(sel_v, [kv],
                               jnp.full((16,), m_val, jnp.float32), mask=lane0)
            plsc.store_scatter(sel_i, [kv], iv, mask=lane0)
            plsc.store_scatter(cand_v, [pv], neg16, mask=lane0)
            lc = p % 16
            if nv > 16:
                col = lane * 16 + lc
                g1 = plsc.load_gather(cand_v, [col])
                g2 = plsc.load_gather(cand_v, [col + 256])
                m1, m2 = jnp.max(g1), jnp.max(g2)
                newm = jnp.maximum(m1, m2)
                r1 = jnp.min(jnp.where(g1 == newm, lane, jnp.int32(_BIG)))
                r2 = jnp.min(jnp.where(g2 == newm, lane + 16, jnp.int32(_BIG)))
                newrow = jnp.where(m1 >= m2, r1, r2)
            else:
                rowc = jnp.minimum(lane, nv - 1)
                g1 = plsc.load_gather(cand_v, [rowc * 16 + lc])
                newm = jnp.max(g1)
                newrow = jnp.min(jnp.where(g1 == newm, rowc, jnp.int32(_BIG)))
            sel = lane == lc
            return (jnp.where(sel, newm, maxv),
                    jnp.where(sel, newrow, rowv), m_val)

        _, _, t_new = lax.fori_loop(0, _K, pick,
                                    (maxv0, rowv0, jnp.float32(0.0)))

        def wipe(q, c):
            cand_v[pl.ds(q * 16, 16)] = neg16
            return c
        lax.fori_loop(0, nv, wipe, 0)
        for m in range(3):
            cand_v[pl.ds(m * 16, 16)] = sel_v[pl.ds(m * 16, 16)]
            cand_i[pl.ds(m * 16, 16)] = sel_i[pl.ds(m * 16, 16)]
        return t_new

    def scan_rows(base, j0, j1):
        for m in range(_NV):
            cand_v[pl.ds(m * 16, 16)] = neg16
        sel_v[pl.ds(32, 16)] = neg16

        def rmpass(j, c):
            b = base + _PAD + j * 200
            m = img[pl.ds(b, 16)]
            for ci in range(1, 12):
                m = jnp.maximum(m, img[pl.ds(b + ci * 16, 16)])
            tail = jnp.where(lane < 8, img[pl.ds(b + 192, 16)], neg16)
            m = jnp.maximum(m, tail)
            rmax[j] = jnp.max(m)
            return c
        lax.fori_loop(j0, j1, rmpass, 0)

        def jbody(j, carry):
            t, cnt_v = carry

            def process():
                cnt_s = jnp.max(cnt_v)
                t2, cnt2 = lax.cond(
                    cnt_s > _CAP - 224,
                    lambda: (reduce_buffer(_NV), jnp.full((16,), _K, jnp.int32)),
                    lambda: (t, cnt_v))
                vrow[pl.ds(0, 16)] = neg16
                b = base + _PAD + j * 200
                ctrs = []
                for ci in range(13):
                    mid = img[pl.ds(b + ci * 16, 16)]
                    cm = jnp.maximum(
                        jnp.maximum(img[pl.ds(b - 200 + ci * 16, 16)], mid),
                        img[pl.ds(b + 200 + ci * 16, 16)])
                    vrow[pl.ds(8 + ci * 16, 16)] = cm
                    ctrs.append((mid, cm))
                vrow[pl.ds(208, 16)] = neg16

                t_v = jnp.full((16,), t2, jnp.float32)

                def hs(zero_phase, cnt3):
                    for ci in range(13):
                        c0 = ci * 16
                        v, ctr = ctrs[ci]
                        lft = vrow[pl.ds(7 + c0, 16)]
                        rgt = vrow[pl.ds(9 + c0, 16)]
                        hm = jnp.maximum(jnp.maximum(lft, ctr), rgt)
                        if zero_phase:
                            val = jnp.where(v == hm, v, jnp.float32(0.0))
                            mask = val >= t_v
                        else:
                            val = v
                            mask = v >= jnp.maximum(hm, t_v)
                        if ci == 12:
                            mask = mask & (lane < 8)
                        npass_v = plsc.all_reduce_population_count(mask)
                        pos = cnt3 - 1 + lax.cumsum(mask.astype(jnp.int32),
                                                    axis=0)
                        plsc.store_scatter(cand_v, [pos], val, mask=mask)
                        plsc.store_scatter(cand_i, [pos],
                                           j * 200 + c0 + lane, mask=mask)
                        cnt3 = cnt3 + npass_v
                    return cnt3

                cnt2 = lax.cond(t2 > 0,
                                lambda c: hs(False, c),
                                lambda c: hs(True, c), cnt2)
                return t2, cnt2

            return lax.cond(rmax[j] >= t, process, lambda: (t, cnt_v))

        lax.fori_loop(j0, j1, jbody,
                      (jnp.float32(0.0), jnp.zeros((16,), jnp.int32)))
        reduce_buffer(_NV)

    def emit_out(r):
        for m in range(3):
            idx = sel_i[pl.ds(m * 16, 16)]
            y = idx // _W
            x = idx - y * _W
            sel_y[pl.ds(m * 16, 16)] = y.astype(jnp.float32)
            sel_x[pl.ds(m * 16, 16)] = x.astype(jnp.float32)
        pltpu.sync_copy(sel_v.at[pl.ds(0, _K)], out_s.at[pl.ds(r * _K, _K)])
        pltpu.sync_copy(sel_i.at[pl.ds(0, _K)], out_i.at[pl.ds(r * _K, _K)])
        pltpu.sync_copy(sel_y.at[pl.ds(0, _K)], out_y.at[pl.ds(r * _K, _K)])
        pltpu.sync_copy(sel_x.at[pl.ds(0, _K)], out_x.at[pl.ds(r * _K, _K)])

    rex = _NWORK * _FULL + cidx + 2 * (sidx // 4)
    qj0 = (sidx % 4) * (_H // 4)

    for m in range(_PAD // 16):
        for base in (0, _IMGW):
            img[pl.ds(base + m * 16, 16)] = neg16
            img[pl.ds(base + _PAD + _N + m * 16, 16)] = neg16
    pltpu.async_copy(heat.at[pl.ds(wid * _N, _N)],
                     img.at[pl.ds(_PAD, _N)], sem.at[0])

    def rloop(m, c):
        r = wid + m * _NWORK
        cur = m % 2
        base = cur * _IMGW
        pltpu.make_async_copy(heat.at[pl.ds(r * _N, _N)],
                              img.at[pl.ds(base + _PAD, _N)],
                              sem.at[cur]).wait()
        nxt = jnp.where(m < _FULL - 1, r + _NWORK, rex)
        pltpu.async_copy(heat.at[pl.ds(nxt * _N, _N)],
                         img.at[pl.ds((_IMGW - base) + _PAD, _N)],
                         sem.at[1 - cur])
        scan_rows(base, 0, _H)
        emit_out(r)
        return c
    lax.fori_loop(0, _FULL, rloop, 0)

    pltpu.make_async_copy(heat.at[pl.ds(rex * _N, _N)],
                          img.at[pl.ds(_PAD, _N)], sem.at[0]).wait()
    scan_rows(0, qj0, qj0 + _H // 4)
    pltpu.sync_copy(sel_v, spart_v.at[pl.ds(sidx * 48, 48)])
    pltpu.sync_copy(sel_i, spart_i.at[pl.ds(sidx * 48, 48)])
    plsc.subcore_barrier()

    @pl.when(sidx % 4 == 0)
    def _():
        for u in range(4):
            pltpu.sync_copy(spart_v.at[pl.ds((sidx + u) * 48, _K)],
                            cand_v.at[pl.ds(u * _K, _K)])
            pltpu.sync_copy(spart_i.at[pl.ds((sidx + u) * 48, _K)],
                            cand_i.at[pl.ds(u * _K, _K)])
        sel_v[pl.ds(32, 16)] = neg16
        reduce_buffer(4 * _K // 16)
        emit_out(rex)


@jax.jit
def _sc_topk(heat1d):
    f32, i32 = jnp.float32, jnp.int32
    out = pl.kernel(
        _sc_body,
        out_type=[jax.ShapeDtypeStruct((_R * _K,), f32),
                  jax.ShapeDtypeStruct((_R * _K,), i32),
                  jax.ShapeDtypeStruct((_R * _K,), f32),
                  jax.ShapeDtypeStruct((_R * _K,), f32)],
        mesh=plsc.VectorSubcoreMesh(core_axis_name="c", subcore_axis_name="s"),
        compiler_params=pltpu.CompilerParams(needs_layout_passes=False),
        scratch_types=[pltpu.VMEM((2 * _IMGW,), f32),
                       pltpu.VMEM((224,), f32),
                       pltpu.SMEM((208,), f32),
                       pltpu.VMEM((_CAP,), f32),
                       pltpu.VMEM((_CAP,), i32),
                       pltpu.VMEM((48,), f32),
                       pltpu.VMEM((48,), i32),
                       pltpu.VMEM((48,), f32),
                       pltpu.VMEM((48,), f32),
                       pltpu.VMEM_SHARED((_NS * 48,), f32),
                       pltpu.VMEM_SHARED((_NS * 48,), i32),
                       pltpu.SemaphoreType.DMA((2,))],
    )(heat1d)
    return tuple(o.reshape(_B, _C, _K) for o in out)


def kernel(heat, K):
    del K
    return _sc_topk(heat.reshape(_R * _N))


# speedup vs baseline: 21.0856x; 1.0249x over previous
import functools

import jax
import jax.numpy as jnp
from jax import lax
from jax.experimental import pallas as pl
from jax.experimental.pallas import tpu as pltpu
from jax.experimental.pallas import tpu_sc as plsc

_B, _C, _H, _W = 8, 17, 200, 200
_K = 40
_R = _B * _C
_N = _H * _W
_CAP = 1024
_NV = _CAP // 16
_NC, _NS = 2, 16
_NWORK = _NC * _NS
_FULL = 4
_PAD = 224
_IMGW = _PAD + _N + _PAD
_NEG = float("-inf")
_BIG = 1 << 30


def _sc_body(heat, out_s, out_i, out_y, out_x,
             img, vrow, rmax, cand_v, cand_i, sel_v, sel_i, sel_y, sel_x,
             spart_v, spart_i, sem):
    sidx = lax.axis_index("s")
    cidx = lax.axis_index("c")
    wid = sidx * _NC + cidx
    lane = lax.iota(jnp.int32, 16)
    lane0 = lane == 0
    neg16 = jnp.full((16,), _NEG, jnp.float32)

    def reduce_buffer(nv):
        def mx(q, carry):
            m, rowv = carry
            vq = cand_v[pl.ds(q * 16, 16)]
            upd = vq > m
            return jnp.maximum(m, vq), jnp.where(upd, q, rowv)
        maxv0, rowv0 = lax.fori_loop(0, nv, mx,
                                     (neg16, jnp.zeros((16,), jnp.int32)))

        def pick(k, carry):
            maxv, rowv, _ = carry
            m_val = jnp.max(maxv)
            p = jnp.min(jnp.where(maxv == m_val, rowv * 16 + lane,
                                  jnp.int32(_BIG)))
            pv = jnp.full((16,), p, jnp.int32)
            iv = plsc.load_gather(cand_i, [pv])
            kv = jnp.full((16,), k, jnp.int32)
            plsc.store_scatter(sel_v, [kv],
                               jnp.full((16,), m_val, jnp.float32), mask=lane0)
            plsc.store_scatter(sel_i, [kv], iv, mask=lane0)
            plsc.store_scatter(cand_v, [pv], neg16, mask=lane0)
            lc = p % 16
            if nv >= 16 and nv % 16 == 0:
                newm = jnp.float32(_NEG)
                newrow = jnp.int32(_BIG)
                for h in range(nv // 16):
                    gh = plsc.load_gather(cand_v, [lane * 16 + h * 256 + lc])
                    mh = jnp.max(gh)
                    rh = jnp.min(jnp.where(gh == mh, lane + h * 16,
                                           jnp.int32(_BIG)))
                    upd = mh > newm
                    newrow = jnp.where(upd, rh, newrow)
                    newm = jnp.maximum(newm, mh)
            else:
                rowc = jnp.minimum(lane, nv - 1)
                g1 = plsc.load_gather(cand_v, [rowc * 16 + lc])
                newm = jnp.max(g1)
                newrow = jnp.min(jnp.where(g1 == newm, rowc, jnp.int32(_BIG)))
            sel = lane == lc
            return (jnp.where(sel, newm, maxv),
                    jnp.where(sel, newrow, rowv), m_val)

        _, _, t_new = lax.fori_loop(0, _K, pick,
                                    (maxv0, rowv0, jnp.float32(0.0)))

        def wipe(q, c):
            cand_v[pl.ds(q * 16, 16)] = neg16
            return c
        lax.fori_loop(0, nv, wipe, 0)
        for m in range(3):
            cand_v[pl.ds(m * 16, 16)] = sel_v[pl.ds(m * 16, 16)]
            cand_i[pl.ds(m * 16, 16)] = sel_i[pl.ds(m * 16, 16)]
        return t_new

    def scan_rows(base, j0, j1):
        for m in range(_NV):
            cand_v[pl.ds(m * 16, 16)] = neg16
        sel_v[pl.ds(32, 16)] = neg16
        vrow[pl.ds(0, 16)] = neg16

        def rmpass(j, c):
            b = base + _PAD + j * 200
            m = img[pl.ds(b, 16)]
            for ci in range(1, 12):
                m = jnp.maximum(m, img[pl.ds(b + ci * 16, 16)])
            tail = jnp.where(lane < 8, img[pl.ds(b + 192, 16)], neg16)
            m = jnp.maximum(m, tail)
            rmax[j] = jnp.max(m)
            return c
        lax.fori_loop(j0, j1, rmpass, 0)

        def jbody(j, carry):
            t, cnt_v = carry

            def process():
                cnt_s = jnp.max(cnt_v)
                t2, cnt2 = lax.cond(
                    cnt_s > _CAP - 224,
                    lambda: (reduce_buffer(_NV), jnp.full((16,), _K, jnp.int32)),
                    lambda: (t, cnt_v))
                b = base + _PAD + j * 200
                ctrs = []
                for ci in range(13):
                    mid = img[pl.ds(b + ci * 16, 16)]
                    cm = jnp.maximum(
                        jnp.maximum(img[pl.ds(b - 200 + ci * 16, 16)], mid),
                        img[pl.ds(b + 200 + ci * 16, 16)])
                    vrow[pl.ds(8 + ci * 16, 16)] = cm
                    ctrs.append((mid, cm))
                vrow[pl.ds(208, 16)] = neg16

                t_v = jnp.full((16,), t2, jnp.float32)

                def hs(zero_phase, cnt3):
                    for ci in range(13):
                        c0 = ci * 16
                        v, ctr = ctrs[ci]
                        lft = vrow[pl.ds(7 + c0, 16)]
                        rgt = vrow[pl.ds(9 + c0, 16)]
                        hm = jnp.maximum(jnp.maximum(lft, ctr), rgt)
                        if zero_phase:
                            val = jnp.where(v == hm, v, jnp.float32(0.0))
                            mask = val >= t_v
                        else:
                            val = v
                            mask = v >= jnp.maximum(hm, t_v)
                        if ci == 12:
                            mask = mask & (lane < 8)
                        npass_v = plsc.all_reduce_population_count(mask)
                        pos = cnt3 - 1 + lax.cumsum(mask.astype(jnp.int32),
                                                    axis=0)
                        plsc.store_scatter(cand_v, [pos], val, mask=mask)
                        plsc.store_scatter(cand_i, [pos],
                                           j * 200 + c0 + lane, mask=mask)
                        cnt3 = cnt3 + npass_v
                    return cnt3

                cnt2 = lax.cond(t2 > 0,
                                lambda c: hs(False, c),
                                lambda c: hs(True, c), cnt2)
                return t2, cnt2

            return lax.cond(rmax[j] >= t, process, lambda: (t, cnt_v))

        lax.fori_loop(j0, j1, jbody,
                      (jnp.float32(0.0), jnp.zeros((16,), jnp.int32)))
        reduce_buffer(_NV)

    def emit_out(r):
        for m in range(3):
            idx = sel_i[pl.ds(m * 16, 16)]
            y = idx // _W
            x = idx - y * _W
            sel_y[pl.ds(m * 16, 16)] = y.astype(jnp.float32)
            sel_x[pl.ds(m * 16, 16)] = x.astype(jnp.float32)
        pltpu.sync_copy(sel_v.at[pl.ds(0, _K)], out_s.at[pl.ds(r * _K, _K)])
        pltpu.sync_copy(sel_i.at[pl.ds(0, _K)], out_i.at[pl.ds(r * _K, _K)])
        pltpu.sync_copy(sel_y.at[pl.ds(0, _K)], out_y.at[pl.ds(r * _K, _K)])
        pltpu.sync_copy(sel_x.at[pl.ds(0, _K)], out_x.at[pl.ds(r * _K, _K)])

    rex = _NWORK * _FULL + cidx + 2 * (sidx // 4)
    qj0 = (sidx % 4) * (_H // 4)

    for m in range(_PAD // 16):
        for base in (0, _IMGW):
            img[pl.ds(base + m * 16, 16)] = neg16
            img[pl.ds(base + _PAD + _N + m * 16, 16)] = neg16
    pltpu.async_copy(heat.at[pl.ds(wid * _N, _N)],
                     img.at[pl.ds(_PAD, _N)], sem.at[0])

    def rloop(m, c):
        r = wid + m * _NWORK
        cur = m % 2
        base = cur * _IMGW
        pltpu.make_async_copy(heat.at[pl.ds(r * _N, _N)],
                              img.at[pl.ds(base + _PAD, _N)],
                              sem.at[cur]).wait()
        nxt = jnp.where(m < _FULL - 1, r + _NWORK, rex)
        pltpu.async_copy(heat.at[pl.ds(nxt * _N, _N)],
                         img.at[pl.ds((_IMGW - base) + _PAD, _N)],
                         sem.at[1 - cur])
        scan_rows(base, 0, _H)
        emit_out(r)
        return c
    lax.fori_loop(0, _FULL, rloop, 0)

    pltpu.make_async_copy(heat.at[pl.ds(rex * _N, _N)],
                          img.at[pl.ds(_PAD, _N)], sem.at[0]).wait()
    scan_rows(0, qj0, qj0 + _H // 4)
    pltpu.sync_copy(sel_v, spart_v.at[pl.ds(sidx * 48, 48)])
    pltpu.sync_copy(sel_i, spart_i.at[pl.ds(sidx * 48, 48)])
    plsc.subcore_barrier()

    @pl.when(sidx % 4 == 0)
    def _():
        for u in range(4):
            pltpu.sync_copy(spart_v.at[pl.ds((sidx + u) * 48, _K)],
                            cand_v.at[pl.ds(u * _K, _K)])
            pltpu.sync_copy(spart_i.at[pl.ds((sidx + u) * 48, _K)],
                            cand_i.at[pl.ds(u * _K, _K)])
        sel_v[pl.ds(32, 16)] = neg16
        reduce_buffer(4 * _K // 16)
        emit_out(rex)


@jax.jit
def _sc_topk(heat1d):
    f32, i32 = jnp.float32, jnp.int32
    out = pl.kernel(
        _sc_body,
        out_type=[jax.ShapeDtypeStruct((_R * _K,), f32),
                  jax.ShapeDtypeStruct((_R * _K,), i32),
                  jax.ShapeDtypeStruct((_R * _K,), f32),
                  jax.ShapeDtypeStruct((_R * _K,), f32)],
        mesh=plsc.VectorSubcoreMesh(core_axis_name="c", subcore_axis_name="s"),
        compiler_params=pltpu.CompilerParams(needs_layout_passes=False),
        scratch_types=[pltpu.VMEM((2 * _IMGW,), f32),
                       pltpu.VMEM((224,), f32),
                       pltpu.SMEM((208,), f32),
                       pltpu.VMEM((_CAP,), f32),
                       pltpu.VMEM((_CAP,), i32),
                       pltpu.VMEM((48,), f32),
                       pltpu.VMEM((48,), i32),
                       pltpu.VMEM((48,), f32),
                       pltpu.VMEM((48,), f32),
                       pltpu.VMEM_SHARED((_NS * 48,), f32),
                       pltpu.VMEM_SHARED((_NS * 48,), i32),
                       pltpu.SemaphoreType.DMA((2,))],
    )(heat1d)
    return tuple(o.reshape(_B, _C, _K) for o in out)


def kernel(heat, K):
    del K
    return _sc_topk(heat.reshape(_R * _N))
